# baseline (speedup 1.0000x reference)
import functools

import jax
import jax.numpy as jnp
from jax import lax
from jax.experimental import pallas as pl
from jax.experimental.pallas import tpu as pltpu
from jax.experimental.pallas import tpu_sc as plsc

D = 8
K = 512
DIM = 512
SUBDIM = DIM // D

_SC_INFO = plsc.get_sparse_core_info()
NC = _SC_INFO.num_cores
NS = _SC_INFO.num_subcores
NW = NC * NS


def _tc_body(x_ref, embt_ref, sn_ref, en_ref, gidx_ref, loss_ref):
    i = pl.program_id(0)
    tn = x_ref.shape[1]
    acc = jnp.float32(0.0)
    iota_kf = lax.broadcasted_iota(jnp.int32, (tn, K), 1).astype(jnp.float32)
    for d in range(D):
        xd = x_ref[0, :, d * SUBDIM:(d + 1) * SUBDIM]
        et = embt_ref[d]
        c = jnp.dot(xd, et, preferred_element_type=jnp.float32)
        sn = sn_ref[0, :, d:d + 1]
        en = en_ref[d:d + 1, :]
        dist = sn + en - 2.0 * c
        mn = jnp.min(dist, axis=1)
        a_f = jnp.min(jnp.where(dist == mn[:, None], iota_kf,
                                jnp.float32(K)), axis=1)
        a = a_f.astype(jnp.int32)
        gidx_ref[0, :, d:d + 1] = (a + d * K)[:, None]
        acc = acc + jnp.sum(mn)

    @pl.when(i == 0)
    def _():
        loss_ref[:, :] = jnp.zeros((1, 1), jnp.float32)

    loss_ref[:, :] += jnp.reshape(acc, (1, 1))


def _sc_gather(table_hbm, gidx_hbm, out_hbm, idx_v, rows_v, sem):
    wid = lax.axis_index("s") * NC + lax.axis_index("c")
    rows_per_w = gidx_hbm.shape[0] // NW
    chunk = rows_v.shape[0]
    base = wid * rows_per_w
    pltpu.sync_copy(gidx_hbm.at[pl.ds(base, rows_per_w)], idx_v)
    for j in range(rows_per_w // chunk):
        off = j * chunk
        pltpu.async_copy(table_hbm.at[idx_v.at[pl.ds(off, chunk)]], rows_v,
                         sem).wait()
        pltpu.sync_copy(rows_v, out_hbm.at[pl.ds(base + off, chunk)])


def _tc_stage(xh, embt, snh, en):
    Bh, T = xh.shape[0], xh.shape[1]
    return pl.pallas_call(
        _tc_body,
        grid=(Bh,),
        in_specs=[
            pl.BlockSpec((1, T, DIM), lambda i: (i, 0, 0)),
            pl.BlockSpec((D, SUBDIM, K), lambda i: (0, 0, 0)),
            pl.BlockSpec((1, T, D), lambda i: (i, 0, 0)),
            pl.BlockSpec((D, K), lambda i: (0, 0)),
        ],
        out_specs=[
            pl.BlockSpec((1, T, D), lambda i: (i, 0, 0)),
            pl.BlockSpec((1, 1), lambda i: (0, 0)),
        ],
        out_shape=[
            jax.ShapeDtypeStruct((Bh, T, D), jnp.int32),
            jax.ShapeDtypeStruct((1, 1), jnp.float32),
        ],
    )(xh, embt, snh, en)


def _sc_stage(table, gidx_flat, chunk):
    rows = gidx_flat.shape[0]
    rows_per_w = rows // NW
    gather = pl.kernel(
        _sc_gather,
        out_type=jax.ShapeDtypeStruct((rows, SUBDIM), jnp.float32),
        mesh=plsc.VectorSubcoreMesh(core_axis_name="c", subcore_axis_name="s"),
        compiler_params=pltpu.CompilerParams(use_tc_tiling_on_sc=False),
        scratch_types=[
            pltpu.VMEM((rows_per_w,), jnp.int32),
            pltpu.VMEM((chunk, SUBDIM), jnp.float32),
            pltpu.SemaphoreType.DMA,
        ],
    )
    return gather(table, gidx_flat)


def kernel(x, embedding):
    B, T, _ = x.shape
    N = B * T
    embt = jnp.transpose(embedding, (0, 2, 1))
    sn = jnp.sum(x.reshape(-1, DIM).reshape(N, D, SUBDIM) ** 2,
                 axis=-1).reshape(B, T, D)
    en = jnp.sum(embedding ** 2, axis=-1)
    table = embedding.reshape(D * K, SUBDIM)

    Bh = B // 2
    gidx_a, loss_a = _tc_stage(x[:Bh], embt, sn[:Bh], en)
    gidx_b, loss_b = _tc_stage(x[Bh:], embt, sn[Bh:], en)
    rows_h = Bh * T * D
    rows_pad = 48384
    pad = jnp.zeros((rows_pad - rows_h,), jnp.int32)
    quant_a = _sc_stage(table,
                        jnp.concatenate([gidx_a.reshape(rows_h), pad]), 72)
    quant_b = _sc_stage(table,
                        jnp.concatenate([gidx_b.reshape(rows_h), pad]), 72)

    recon = jnp.concatenate([quant_a[:rows_h].reshape(Bh, T, DIM),
                             quant_b[:rows_h].reshape(Bh, T, DIM)], axis=0)
    iota_d = jnp.arange(D, dtype=jnp.int32) * K
    indices = (jnp.concatenate([gidx_a, gidx_b], axis=0)
               - iota_d[None, None, :])
    vq_loss = ((loss_a[0, 0] + loss_b[0, 0])
               * (1.25 / (N * DIM))).astype(jnp.float32)
    return recon, indices, vq_loss


# speedup vs baseline: 1.5234x; 1.5234x over previous
import functools

import jax
import jax.numpy as jnp
from jax import lax
from jax.experimental import pallas as pl
from jax.experimental.pallas import tpu as pltpu
from jax.experimental.pallas import tpu_sc as plsc

D = 8
K = 512
DIM = 512
SUBDIM = DIM // D

_SC_INFO = plsc.get_sparse_core_info()
NC = _SC_INFO.num_cores
NS = _SC_INFO.num_subcores
NW = NC * NS


def _tc_body(x_ref, embt_ref, sn_ref, en_ref, gidx_ref, loss_ref):
    i = pl.program_id(0)
    tn = x_ref.shape[1]
    acc = jnp.float32(0.0)
    iota_kf = lax.broadcasted_iota(jnp.int32, (tn, K), 1).astype(jnp.float32)
    for d in range(D):
        xd = x_ref[0, :, d * SUBDIM:(d + 1) * SUBDIM]
        et = embt_ref[d]
        c = jnp.dot(xd, et, preferred_element_type=jnp.float32)
        sn = sn_ref[0, :, d:d + 1]
        en = en_ref[d:d + 1, :]
        dist = sn + en - 2.0 * c
        mn = jnp.min(dist, axis=1)
        a_f = jnp.min(jnp.where(dist == mn[:, None], iota_kf,
                                jnp.float32(K)), axis=1)
        a = a_f.astype(jnp.int32)
        gidx_ref[0, :, d:d + 1] = (a + d * K)[:, None]
        acc = acc + jnp.sum(mn)

    @pl.when(i == 0)
    def _():
        loss_ref[:, :] = jnp.zeros((1, 1), jnp.float32)

    loss_ref[:, :] += jnp.reshape(acc, (1, 1))


def _sc_gather(table_hbm, gidx_hbm, out_hbm, idx_v, rows_v0, rows_v1,
               sg0, sg1, sw0, sw1):
    wid = lax.axis_index("s") * NC + lax.axis_index("c")
    rows_per_w = gidx_hbm.shape[0] // NW
    chunk = rows_v0.shape[0]
    nb = rows_per_w // chunk
    base = wid * rows_per_w
    bufs = (rows_v0, rows_v1)
    sgs = (sg0, sg1)
    sws = (sw0, sw1)
    pltpu.sync_copy(gidx_hbm.at[pl.ds(base, rows_per_w)], idx_v)

    def start_gather(j):
        b = bufs[j & 1]
        return pltpu.async_copy(
            table_hbm.at[idx_v.at[pl.ds(j * chunk, chunk)]], b, sgs[j & 1])

    gh = {0: start_gather(0)}
    if nb > 1:
        gh[1] = start_gather(1)
    for j in range(nb):
        b = j & 1
        gh[j].wait()
        wh = pltpu.async_copy(bufs[b], out_hbm.at[pl.ds(base + j * chunk,
                                                        chunk)], sws[b])
        if j + 2 < nb:
            wh.wait()
            gh[j + 2] = start_gather(j + 2)
        else:
            wh.wait()


def _tc_stage(xh, embt, snh, en):
    Bh, T = xh.shape[0], xh.shape[1]
    return pl.pallas_call(
        _tc_body,
        grid=(Bh,),
        in_specs=[
            pl.BlockSpec((1, T, DIM), lambda i: (i, 0, 0)),
            pl.BlockSpec((D, SUBDIM, K), lambda i: (0, 0, 0)),
            pl.BlockSpec((1, T, D), lambda i: (i, 0, 0)),
            pl.BlockSpec((D, K), lambda i: (0, 0)),
        ],
        out_specs=[
            pl.BlockSpec((1, T, D), lambda i: (i, 0, 0)),
            pl.BlockSpec((1, 1), lambda i: (0, 0)),
        ],
        out_shape=[
            jax.ShapeDtypeStruct((Bh, T, D), jnp.int32),
            jax.ShapeDtypeStruct((1, 1), jnp.float32),
        ],
    )(xh, embt, snh, en)


def _sc_stage(table, gidx_flat, chunk):
    rows = gidx_flat.shape[0]
    rows_per_w = rows // NW
    gather = pl.kernel(
        _sc_gather,
        out_type=jax.ShapeDtypeStruct((rows, SUBDIM), jnp.float32),
        mesh=plsc.VectorSubcoreMesh(core_axis_name="c", subcore_axis_name="s"),
        compiler_params=pltpu.CompilerParams(use_tc_tiling_on_sc=False),
        scratch_types=[
            pltpu.VMEM((rows_per_w,), jnp.int32),
            pltpu.VMEM((chunk, SUBDIM), jnp.float32),
            pltpu.VMEM((chunk, SUBDIM), jnp.float32),
            pltpu.SemaphoreType.DMA,
            pltpu.SemaphoreType.DMA,
            pltpu.SemaphoreType.DMA,
            pltpu.SemaphoreType.DMA,
        ],
    )
    return gather(table, gidx_flat)


def kernel(x, embedding):
    B, T, _ = x.shape
    N = B * T
    embt = jnp.transpose(embedding, (0, 2, 1))
    sn = jnp.sum(x.reshape(-1, DIM).reshape(N, D, SUBDIM) ** 2,
                 axis=-1).reshape(B, T, D)
    en = jnp.sum(embedding ** 2, axis=-1)
    table = embedding.reshape(D * K, SUBDIM)

    gidx, loss = _tc_stage(x, embt, sn, en)
    quant = _sc_stage(table, gidx.reshape(N * D), 120)

    recon = quant.reshape(B, T, DIM)
    iota_d = jnp.arange(D, dtype=jnp.int32) * K
    indices = gidx - iota_d[None, None, :]
    vq_loss = (loss[0, 0] * (1.25 / (N * DIM))).astype(jnp.float32)
    return recon, indices, vq_loss
